# baseline (speedup 1.0000x reference)
import jax
import jax.numpy as jnp
import numpy as np
from jax.experimental import pallas as pl
from jax.experimental.pallas import tpu as pltpu

_ANCHORS = [1.3221, 1.73145, 3.19275, 4.00944, 5.05587,
            8.09892, 9.47112, 4.84053, 11.2364, 10.0071]
_A = 5
_G = 19
_HW = _G * _G
_N = _A * _HW
_NP = 1920
_NC = 2048
_RT = 128
_CW = 512
_NGT = 50
_NGTP = 64
_CONF = 0.5
_NMS_T = 0.45
_IOU_T = 0.5
_CAN = _NMS_T / (1.0 + _NMS_T)
_CAG = _IOU_T / (1.0 + _IOU_T)
_SLABS = [(0, 512), (512, 1024), (1024, 1536), (1536, 1920)]


def _sig(x):
    return 1.0 / (1.0 + jnp.exp(-x))


def _nms_body(chr_ref, chc_ref, tgt_ref, cc_ref, out_ref,
              s_scr, ccol_scr, crow_scr, k_scr, sup_scr, id_scr):
    f32 = jnp.float32
    bf16 = jnp.bfloat16
    pid = pl.program_id(0)

    @pl.when(pid == 0)
    def _init():
        ri = jax.lax.broadcasted_iota(jnp.int32, (_CW, _CW), 0)
        ci = jax.lax.broadcasted_iota(jnp.int32, (_CW, _CW), 1)
        id_scr[...] = (ri == ci).astype(f32)
        ccol_scr[_NP:_NC, :] = jnp.zeros((_NC - _NP, 8), f32)

    cc = cc_ref[...]
    tc = chc_ref[0]
    x_c = (_sig(tc[:, 0:1]) + cc[:, 0:1]) / 19.0
    y_c = (_sig(tc[:, 1:2]) + cc[:, 1:2]) / 19.0
    w_c = jnp.exp(tc[:, 2:3]) * cc[:, 2:3] / 19.0
    h_c = jnp.exp(tc[:, 3:4]) * cc[:, 3:4] / 19.0
    conf_c = _sig(tc[:, 4:5])
    area_c = w_c * h_c
    pcol = jnp.concatenate([
        x_c - w_c / 2.0,
        x_c + w_c / 2.0,
        y_c - h_c / 2.0,
        y_c + h_c / 2.0,
        area_c * _CAN,
        area_c * _CAG,
        conf_c,
        jnp.zeros((_NP, 1), f32),
    ], axis=1)

    ch = chr_ref[0]
    vf = (_sig(ch[4:5]) > _CONF).astype(f32)
    incl = vf
    d = 1
    while d < _NP:
        incl = incl + jnp.concatenate(
            [jnp.zeros((1, d), f32), incl[:, :_NP - d]], axis=1)
        d *= 2
    pos = incl - vf
    m_f = jnp.sum(vf)
    m_i = m_f.astype(jnp.int32)

    m_io = jax.lax.broadcasted_iota(jnp.int32, (_NP, 1), 0).astype(f32)
    acc = jnp.zeros((_NP, 8), f32)
    for (a, b) in _SLABS:
        p_sl = ((m_io == pos[:, a:b]) & (vf[:, a:b] > 0.5)).astype(f32)
        acc = acc + jax.lax.dot_general(
            p_sl, pcol[a:b, :], (((1,), (0,)), ((), ())),
            precision=jax.lax.Precision.HIGHEST,
            preferred_element_type=f32)
    ccol_scr[0:_NP, :] = acc

    for jb in range(_NC // _CW):
        @pl.when(jb * _CW < m_i)
        def _tr(jb=jb):
            blk = ccol_scr[jb * _CW:(jb + 1) * _CW, :]
            crow_scr[:, jb * _CW:(jb + 1) * _CW] = jax.lax.dot_general(
                blk, id_scr[...], (((0,), (0,)), ((), ())),
                precision=jax.lax.Precision.HIGHEST,
                preferred_element_type=f32)

    for tr in range(_NP // _RT):
        @pl.when(tr * _RT < m_i)
        def _row(tr=tr):
            lo = tr * _RT
            blk = ccol_scr[lo:lo + _RT, :]
            l1, r1 = blk[:, 0:1], blk[:, 1:2]
            t1, b1 = blk[:, 2:3], blk[:, 3:4]
            ca1, c1 = blk[:, 4:5], blk[:, 6:7]
            for jc in range(_NC // _CW):
                @pl.when(jc * _CW < m_i)
                def _col(jc=jc, lo=lo, l1=l1, r1=r1, t1=t1, b1=b1,
                         ca1=ca1, c1=c1):
                    clo = jc * _CW
                    l2 = crow_scr[0:1, clo:clo + _CW]
                    r2 = crow_scr[1:2, clo:clo + _CW]
                    t2 = crow_scr[2:3, clo:clo + _CW]
                    b2 = crow_scr[3:4, clo:clo + _CW]
                    ca2 = crow_scr[4:5, clo:clo + _CW]
                    c2 = crow_scr[6:7, clo:clo + _CW]
                    cw = jnp.maximum(jnp.minimum(r1, r2) - jnp.maximum(l1, l2), 0.0)
                    chh = jnp.maximum(jnp.minimum(b1, b2) - jnp.maximum(t1, t2), 0.0)
                    overl = cw * chh > ca1 + ca2
                    hi, chi = lo + _RT, clo + _CW
                    if chi <= lo:
                        s = (c1 > c2) & overl
                        s_scr[lo:hi, clo:chi] = s.astype(bf16)
                    elif hi <= clo:
                        s = (c1 >= c2) & overl
                        s_scr[lo:hi, clo:chi] = s.astype(bf16)
                    else:
                        if lo > clo:
                            s = (c1 > c2[:, :lo - clo]) & overl[:, :lo - clo]
                            s_scr[lo:hi, clo:lo] = s.astype(bf16)
                        ii = jax.lax.broadcasted_iota(jnp.int32, (_RT, 1), 0)
                        jj = jax.lax.broadcasted_iota(jnp.int32, (1, _RT), 1)
                        cm = c2[:, lo - clo:hi - clo]
                        rank = (c1 > cm) | ((c1 >= cm) & (ii < jj))
                        s = rank & overl[:, lo - clo:hi - clo]
                        s_scr[lo:hi, lo:hi] = s.astype(bf16)
                        if chi > hi:
                            s = (c1 >= c2[:, hi - clo:]) & overl[:, hi - clo:]
                            s_scr[lo:hi, hi:chi] = s.astype(bf16)

    lane = jax.lax.broadcasted_iota(jnp.int32, (1, _NC), 1)
    v = lane < m_i
    n_t = (m_i + (_RT - 1)) >> 7

    def sweep(k):
        k_scr[0:1, :] = k.astype(bf16)
        for jb in range(_NC // _CW):
            @pl.when(jb * _CW < m_i)
            def _mv(jb=jb):
                def kb_body(kb, a):
                    ks = k_scr[0:1, pl.ds(kb * _RT, _RT)]
                    sb = s_scr[pl.ds(kb * _RT, _RT), jb * _CW:(jb + 1) * _CW]
                    return a + jax.lax.dot_general(
                        ks, sb, (((1,), (0,)), ((), ())),
                        preferred_element_type=f32)
                acc = jax.lax.fori_loop(0, n_t, kb_body,
                                        jnp.zeros((1, _CW), f32))
                sup_scr[0:1, jb * _CW:(jb + 1) * _CW] = acc
        sup = sup_scr[0:1, :]
        return jnp.where(v & (sup < 0.5), 1.0, 0.0)

    def cond(c):
        old, new = c
        return jnp.any(old != new)

    def body(c):
        _, k = c
        return (k, sweep(k))

    k0 = v.astype(f32)
    _, keep = jax.lax.while_loop(cond, body, (k0, sweep(k0)))

    cl = crow_scr[0:1, :]
    crr = crow_scr[1:2, :]
    ct = crow_scr[2:3, :]
    cb = crow_scr[3:4, :]
    cagr = crow_scr[5:6, :]
    tg = tgt_ref[0]
    gx, gy, gw, gh = tg[:, 1:2], tg[:, 2:3], tg[:, 3:4], tg[:, 4:5]
    gcw = jnp.maximum(jnp.minimum(gx + gw / 2.0, crr)
                      - jnp.maximum(gx - gw / 2.0, cl), 0.0)
    gch = jnp.maximum(jnp.minimum(gy + gh / 2.0, cb)
                      - jnp.maximum(gy - gh / 2.0, ct), 0.0)
    hit = (keep > 0.5) & (gcw * gch > (gw * gh) * _CAG + cagr)
    anyhit = jnp.max(hit.astype(f32), axis=1, keepdims=True)

    ind = (gx != 0).astype(f32)
    row_i = jax.lax.broadcasted_iota(jnp.int32, (_NGTP, _NGTP), 0)
    col_i = jax.lax.broadcasted_iota(jnp.int32, (_NGTP, _NGTP), 1)
    lower = (col_i <= row_i).astype(f32)
    counts = jax.lax.dot_general(
        lower, ind, (((1,), (0,)), ((), ())), preferred_element_type=f32)
    gnum = jax.lax.broadcasted_iota(jnp.int32, (_NGTP, 1), 0).astype(f32) + 1.0
    gvalid = counts == gnum

    t_sum = jnp.sum(gvalid.astype(f32))
    c_sum = jnp.sum((gvalid & (anyhit > 0.5)).astype(f32))
    p_sum = jnp.sum(keep)

    olane = jax.lax.broadcasted_iota(jnp.int32, (1, 128), 1)
    out_ref[0] = (jnp.where(olane == 0, t_sum, 0.0)
                  + jnp.where(olane == 1, p_sum, 0.0)
                  + jnp.where(olane == 2, c_sum, 0.0))


def kernel(output, target):
    f32 = jnp.float32
    B = output.shape[0]
    out5 = output.reshape(B, _A, 5 + 20, _HW)[:, :, :5, :]
    chr_ = out5.transpose(0, 2, 1, 3).reshape(B, 5, _N)
    chr_ = jnp.pad(chr_, ((0, 0), (0, 0), (0, _NP - _N)))
    chc_ = out5.transpose(0, 1, 3, 2).reshape(B, _N, 5)
    chc_ = jnp.pad(chc_, ((0, 0), (0, _NP - _N), (0, 3)))
    tgt = jnp.pad(target.reshape(B, _NGT, 5), ((0, 0), (0, _NGTP - _NGT), (0, 3)))

    hw = np.arange(_HW)
    cc = np.zeros((_NP, 8), np.float32)
    cc[:_N, 0] = np.tile(hw % _G, _A)
    cc[:_N, 1] = np.tile(hw // _G, _A)
    cc[:_N, 2] = np.repeat(np.asarray(_ANCHORS[0::2], np.float32), _HW)
    cc[:_N, 3] = np.repeat(np.asarray(_ANCHORS[1::2], np.float32), _HW)
    cc[_N:, 2:4] = 1.0

    partial = pl.pallas_call(
        _nms_body,
        grid=(B,),
        in_specs=[
            pl.BlockSpec((1, 5, _NP), lambda i: (i, 0, 0)),
            pl.BlockSpec((1, _NP, 8), lambda i: (i, 0, 0)),
            pl.BlockSpec((1, _NGTP, 8), lambda i: (i, 0, 0)),
            pl.BlockSpec((_NP, 8), lambda i: (0, 0)),
        ],
        out_specs=pl.BlockSpec((1, 1, 128), lambda i: (i, 0, 0)),
        out_shape=jax.ShapeDtypeStruct((B, 1, 128), f32),
        scratch_shapes=[
            pltpu.VMEM((_NP, _NC), jnp.bfloat16),
            pltpu.VMEM((_NC, 8), f32),
            pltpu.VMEM((8, _NC), f32),
            pltpu.VMEM((8, _NC), jnp.bfloat16),
            pltpu.VMEM((8, _NC), f32),
            pltpu.VMEM((_CW, _CW), f32),
        ],
    )(chr_, chc_, tgt, jnp.asarray(cc))
    return jnp.sum(partial[:, 0, :3], axis=0)


# speedup vs baseline: 1.7891x; 1.7891x over previous
import jax
import jax.numpy as jnp
import numpy as np
from jax.experimental import pallas as pl
from jax.experimental.pallas import tpu as pltpu

_ANCHORS = [1.3221, 1.73145, 3.19275, 4.00944, 5.05587,
            8.09892, 9.47112, 4.84053, 11.2364, 10.0071]
_A = 5
_G = 19
_HW = _G * _G
_N = _A * _HW
_NP = 1920
_RT = 128
_TILES = _NP // _RT
_NGT = 50
_NGTP = 64
_CONF = 0.5
_NMS_T = 0.45
_IOU_T = 0.5


def _sig(x):
    return 1.0 / (1.0 + jnp.exp(-x))


def _nms_body(chr_ref, chc_ref, tgt_ref, cr_ref, cc_ref, out_ref, s_scr):
    f32 = jnp.float32
    cr = cr_ref[...]
    ch = chr_ref[0]
    x_r = (_sig(ch[0:1]) + cr[0:1]) / 19.0
    y_r = (_sig(ch[1:2]) + cr[1:2]) / 19.0
    w_r = jnp.exp(ch[2:3]) * cr[2:3] / 19.0
    h_r = jnp.exp(ch[3:4]) * cr[3:4] / 19.0
    conf_r = _sig(ch[4:5])
    v_r = conf_r > _CONF
    l_r = x_r - w_r / 2.0
    r_r = x_r + w_r / 2.0
    t_r = y_r - h_r / 2.0
    b_r = y_r + h_r / 2.0
    area_r = w_r * h_r
    can_r = area_r * (_NMS_T / (1.0 + _NMS_T))
    cag_r = area_r * (_IOU_T / (1.0 + _IOU_T))

    cc = cc_ref[...]
    tc = chc_ref[0]
    x_c = (_sig(tc[:, 0:1]) + cc[:, 0:1]) / 19.0
    y_c = (_sig(tc[:, 1:2]) + cc[:, 1:2]) / 19.0
    w_c = jnp.exp(tc[:, 2:3]) * cc[:, 2:3] / 19.0
    h_c = jnp.exp(tc[:, 3:4]) * cc[:, 3:4] / 19.0
    conf_c = _sig(tc[:, 4:5])
    l_c = x_c - w_c / 2.0
    r_c = x_c + w_c / 2.0
    t_c = y_c - h_c / 2.0
    b_c = y_c + h_c / 2.0
    can_c = (w_c * h_c) * (_NMS_T / (1.0 + _NMS_T))

    def build_tile(off):
        bf16 = jnp.bfloat16
        sl = lambda v: v[off:off + _RT]
        l1, r1, t1, b1 = sl(l_c), sl(r_c), sl(t_c), sl(b_c)
        c1, ca1 = sl(conf_c), sl(can_c)
        cw = jnp.maximum(jnp.minimum(r1, r_r) - jnp.maximum(l1, l_r), 0.0)
        chh = jnp.maximum(jnp.minimum(b1, b_r) - jnp.maximum(t1, t_r), 0.0)
        overl = cw * chh > ca1 + can_r
        hi = off + _RT
        if off > 0:
            s = (c1 > conf_r[:, :off]) & overl[:, :off]
            s_scr[off:hi, 0:off] = s.astype(bf16)
        ii = jax.lax.broadcasted_iota(jnp.int32, (_RT, 1), 0)
        jj = jax.lax.broadcasted_iota(jnp.int32, (1, _RT), 1)
        cm = conf_r[:, off:hi]
        rank = (c1 > cm) | ((c1 >= cm) & (ii < jj))
        s_scr[off:hi, off:hi] = (rank & overl[:, off:hi]).astype(bf16)
        if hi < _NP:
            s = (c1 >= conf_r[:, hi:]) & overl[:, hi:]
            s_scr[off:hi, hi:] = s.astype(bf16)

    for t in range(_TILES):
        build_tile(t * _RT)

    v_f = v_r.astype(f32)

    def step(k):
        sup = jax.lax.dot_general(
            k.astype(jnp.bfloat16), s_scr[...],
            (((1,), (0,)), ((), ())), preferred_element_type=f32)
        return jnp.where(v_r & (sup < 0.5), 1.0, 0.0)

    def cond(c):
        old, new = c
        return jnp.any(old != new)

    def body(c):
        _, k = c
        return (k, step(k))

    _, keep = jax.lax.while_loop(cond, body, (v_f, step(v_f)))

    tg = tgt_ref[0]
    gx, gy, gw, gh = tg[:, 1:2], tg[:, 2:3], tg[:, 3:4], tg[:, 4:5]
    cw = jnp.maximum(
        jnp.minimum(gx + gw / 2.0, r_r) - jnp.maximum(gx - gw / 2.0, l_r), 0.0)
    chh = jnp.maximum(
        jnp.minimum(gy + gh / 2.0, b_r) - jnp.maximum(gy - gh / 2.0, t_r), 0.0)
    cag = (gw * gh) * (_IOU_T / (1.0 + _IOU_T))
    hit = (keep > 0.5) & (cw * chh > cag + cag_r)
    anyhit = jnp.max(hit.astype(f32), axis=1, keepdims=True)

    ind = (gx != 0).astype(f32)
    row_i = jax.lax.broadcasted_iota(jnp.int32, (_NGTP, _NGTP), 0)
    col_i = jax.lax.broadcasted_iota(jnp.int32, (_NGTP, _NGTP), 1)
    lower = (col_i <= row_i).astype(f32)
    counts = jax.lax.dot_general(
        lower, ind, (((1,), (0,)), ((), ())), preferred_element_type=f32)
    gnum = jax.lax.broadcasted_iota(jnp.int32, (_NGTP, 1), 0).astype(f32) + 1.0
    gvalid = counts == gnum

    t_sum = jnp.sum(gvalid.astype(f32))
    c_sum = jnp.sum((gvalid & (anyhit > 0.5)).astype(f32))
    p_sum = jnp.sum(keep)

    lane = jax.lax.broadcasted_iota(jnp.int32, (1, 128), 1)
    out_ref[0] = (jnp.where(lane == 0, t_sum, 0.0)
                  + jnp.where(lane == 1, p_sum, 0.0)
                  + jnp.where(lane == 2, c_sum, 0.0))


def kernel(output, target):
    f32 = jnp.float32
    B = output.shape[0]
    out5 = output.reshape(B, _A, 5 + 20, _HW)[:, :, :5, :]
    chr_ = out5.transpose(0, 2, 1, 3).reshape(B, 5, _N)
    chr_ = jnp.pad(chr_, ((0, 0), (0, 0), (0, _NP - _N)))
    chc_ = out5.transpose(0, 1, 3, 2).reshape(B, _N, 5)
    chc_ = jnp.pad(chc_, ((0, 0), (0, _NP - _N), (0, 3)))
    tgt = jnp.pad(target.reshape(B, _NGT, 5), ((0, 0), (0, _NGTP - _NGT), (0, 3)))

    hw = np.arange(_HW)
    cr = np.zeros((8, _NP), np.float32)
    cr[0, :_N] = np.tile(hw % _G, _A)
    cr[1, :_N] = np.tile(hw // _G, _A)
    cr[2, :_N] = np.repeat(np.asarray(_ANCHORS[0::2], np.float32), _HW)
    cr[3, :_N] = np.repeat(np.asarray(_ANCHORS[1::2], np.float32), _HW)
    cr[2:4, _N:] = 1.0
    cc = np.ascontiguousarray(cr.T)

    partial = pl.pallas_call(
        _nms_body,
        grid=(B,),
        in_specs=[
            pl.BlockSpec((1, 5, _NP), lambda i: (i, 0, 0)),
            pl.BlockSpec((1, _NP, 8), lambda i: (i, 0, 0)),
            pl.BlockSpec((1, _NGTP, 8), lambda i: (i, 0, 0)),
            pl.BlockSpec((8, _NP), lambda i: (0, 0)),
            pl.BlockSpec((_NP, 8), lambda i: (0, 0)),
        ],
        out_specs=pl.BlockSpec((1, 1, 128), lambda i: (i, 0, 0)),
        out_shape=jax.ShapeDtypeStruct((B, 1, 128), f32),
        scratch_shapes=[pltpu.VMEM((_NP, _NP), jnp.bfloat16)],
    )(chr_, chc_, tgt, jnp.asarray(cr), jnp.asarray(cc))
    return jnp.sum(partial[:, 0, :3], axis=0)
